# Initial kernel scaffold; baseline (speedup 1.0000x reference)
import jax
import jax.numpy as jnp
from jax.experimental import pallas as pl


def kernel(input_points, supernode_idxs, W_in, b_in, W1, b1, W2, b2):
    raise NotImplementedError("write your pallas kernel here")


# speedup vs baseline: 5.1342x; 5.1342x over previous
import functools
import numpy as np
import jax
import jax.numpy as jnp
from jax import lax
from jax.experimental import pallas as pl
from jax.experimental.pallas import tpu as pltpu

HIDDEN = 256
NDIM = 3
K = 32
SBLK = 256


def _posembed_consts():
    dim_per = HIDDEN // NDIM
    if dim_per % 2 == 1:
        dim_per -= 1
    half = dim_per // 2
    omega = 1.0 / (10000.0 ** (np.arange(half, dtype=np.float32) / half))
    D = np.zeros((NDIM, HIDDEN), dtype=np.float32)
    sinmask = np.zeros((1, HIDDEN), dtype=np.float32)
    valid = np.zeros((1, HIDDEN), dtype=np.float32)
    for j in range(NDIM * dim_per):
        d, r = j // dim_per, j % dim_per
        w = omega[r] if r < half else omega[r - half]
        D[d, j] = w
        sinmask[0, j] = 1.0 if r < half else 0.0
        valid[0, j] = 1.0
    return jnp.asarray(D), jnp.asarray(sinmask), jnp.asarray(valid)


def _body(x_ref, xt_ref, sidx_ref, dmat_ref, sinm_ref, valid_ref,
          win_ref, bin_ref, w1_ref, b1_ref, w2_ref, b2_ref,
          out_ref, y_scr, *, n_points):
    s_blk = pl.program_id(1)
    f32 = jnp.float32

    @pl.when(s_blk == 0)
    def _compute_mlp():
        xx = x_ref[0]
        proj = jnp.dot(xx, win_ref[...], preferred_element_type=f32) + bin_ref[...]
        t = jnp.dot(xx, dmat_ref[...], preferred_element_type=f32)
        pos = jnp.where(sinm_ref[...] > 0, jnp.sin(t), jnp.cos(t)) * valid_ref[...]
        h = proj + pos
        h = jnp.dot(h, w1_ref[...], preferred_element_type=f32) + b1_ref[...]
        h = jax.nn.gelu(h)
        y_scr[...] = jnp.dot(h, w2_ref[...], preferred_element_type=f32) + b2_ref[...]

    sidx = sidx_ref[0]
    iota = lax.broadcasted_iota(jnp.int32, (SBLK, n_points), 1)
    onehot_s = (iota == sidx).astype(f32)
    sup = jnp.dot(onehot_s, x_ref[0], preferred_element_type=f32)

    xt = xt_ref[0]
    acc = jnp.zeros((SBLK, n_points), dtype=f32)
    for d in range(NDIM):
        diff = sup[:, d:d + 1] - xt[d:d + 1, :]
        acc = acc + diff * diff

    def body(_, carry):
        acc, A = carry
        m = jnp.min(acc, axis=1, keepdims=True)
        cand = jnp.where(acc == m, iota, jnp.int32(n_points))
        first = jnp.min(cand, axis=1, keepdims=True)
        mask = iota == first
        A = A + mask.astype(f32)
        acc = jnp.where(mask, jnp.float32(jnp.inf), acc)
        return acc, A

    A0 = jnp.zeros((SBLK, n_points), dtype=f32)
    _, A = lax.fori_loop(0, K, body, (acc, A0))

    out_ref[0] = jnp.dot(A, y_scr[...], preferred_element_type=f32) * f32(1.0 / K)


def kernel(input_points, supernode_idxs, W_in, b_in, W1, b1, W2, b2):
    B, N, _ = input_points.shape
    S = supernode_idxs.shape[1]
    dmat, sinmask, valid = _posembed_consts()
    x = input_points.astype(jnp.float32)
    xt = jnp.transpose(x, (0, 2, 1))
    sidx = supernode_idxs.astype(jnp.int32).reshape(B, S, 1)

    grid = (B, S // SBLK)
    out = pl.pallas_call(
        functools.partial(_body, n_points=N),
        grid=grid,
        in_specs=[
            pl.BlockSpec((1, N, NDIM), lambda b, s: (b, 0, 0)),
            pl.BlockSpec((1, NDIM, N), lambda b, s: (b, 0, 0)),
            pl.BlockSpec((1, SBLK, 1), lambda b, s: (b, s, 0)),
            pl.BlockSpec((NDIM, HIDDEN), lambda b, s: (0, 0)),
            pl.BlockSpec((1, HIDDEN), lambda b, s: (0, 0)),
            pl.BlockSpec((1, HIDDEN), lambda b, s: (0, 0)),
            pl.BlockSpec((NDIM, HIDDEN), lambda b, s: (0, 0)),
            pl.BlockSpec((1, HIDDEN), lambda b, s: (0, 0)),
            pl.BlockSpec((HIDDEN, HIDDEN), lambda b, s: (0, 0)),
            pl.BlockSpec((1, HIDDEN), lambda b, s: (0, 0)),
            pl.BlockSpec((HIDDEN, HIDDEN), lambda b, s: (0, 0)),
            pl.BlockSpec((1, HIDDEN), lambda b, s: (0, 0)),
        ],
        out_specs=pl.BlockSpec((1, SBLK, HIDDEN), lambda b, s: (b, s, 0)),
        out_shape=jax.ShapeDtypeStruct((B, S, HIDDEN), jnp.float32),
        scratch_shapes=[pltpu.VMEM((N, HIDDEN), jnp.float32)],
    )(x, xt, sidx, dmat, sinmask, valid,
      W_in, b_in.reshape(1, HIDDEN), W1, b1.reshape(1, HIDDEN),
      W2, b2.reshape(1, HIDDEN))
    return out


# speedup vs baseline: 11.2001x; 2.1814x over previous
import functools
import numpy as np
import jax
import jax.numpy as jnp
from jax import lax
from jax.experimental import pallas as pl
from jax.experimental.pallas import tpu as pltpu

HIDDEN = 256
NDIM = 3
K = 32
SBLK = 256


def _posembed_consts():
    dim_per = HIDDEN // NDIM
    if dim_per % 2 == 1:
        dim_per -= 1
    half = dim_per // 2
    omega = 1.0 / (10000.0 ** (np.arange(half, dtype=np.float32) / half))
    D = np.zeros((NDIM, HIDDEN), dtype=np.float32)
    phase = np.zeros((1, HIDDEN), dtype=np.float32)
    valid = np.zeros((1, HIDDEN), dtype=np.float32)
    for j in range(NDIM * dim_per):
        d, r = j // dim_per, j % dim_per
        w = omega[r] if r < half else omega[r - half]
        D[d, j] = w
        phase[0, j] = 0.0 if r < half else np.float32(np.pi / 2)
        valid[0, j] = 1.0
    return jnp.asarray(D), jnp.asarray(phase), jnp.asarray(valid)


def _body(x_ref, xt_ref, sidx_ref, dmat_ref, sinm_ref, valid_ref,
          win_ref, bin_ref, w1_ref, b1_ref, w2_ref, b2_ref,
          out_ref, y_scr, *, n_points):
    s_blk = pl.program_id(1)
    f32 = jnp.float32

    @pl.when(s_blk == 0)
    def _compute_mlp():
        xx = x_ref[0]
        proj = jnp.dot(xx, win_ref[...], preferred_element_type=f32) + bin_ref[...]
        t = jnp.dot(xx, dmat_ref[...], preferred_element_type=f32)
        pos = jnp.sin(t + sinm_ref[...]) * valid_ref[...]
        h = proj + pos
        h = jnp.dot(h, w1_ref[...], preferred_element_type=f32) + b1_ref[...]
        h = jax.nn.gelu(h)
        y_scr[...] = jnp.dot(h, w2_ref[...], preferred_element_type=f32) + b2_ref[...]

    sidx = sidx_ref[0]
    iota = lax.broadcasted_iota(jnp.int32, (SBLK, n_points), 1)
    onehot_s = (iota == sidx).astype(f32)
    sup = jnp.dot(onehot_s, x_ref[0], preferred_element_type=f32)

    xt = xt_ref[0]
    acc = jnp.zeros((SBLK, n_points), dtype=f32)
    for d in range(NDIM):
        diff = sup[:, d:d + 1] - xt[d:d + 1, :]
        acc = acc + diff * diff

    bits = lax.bitcast_convert_type(acc, jnp.int32)
    lo = jnp.zeros((SBLK, 1), jnp.int32)
    hi = jnp.max(bits, axis=1, keepdims=True)

    def bs_body(_, carry):
        lo, hi = carry
        mid = lo + (hi - lo) // 2
        cnt = jnp.sum((bits <= mid).astype(jnp.int32), axis=1, keepdims=True)
        ge = cnt >= K
        return jnp.where(ge, lo, mid + 1), jnp.where(ge, mid, hi)

    lo, hi = lax.fori_loop(0, 31, bs_body, (lo, hi))
    v_kth = lo
    lt = bits < v_kth
    eq = bits == v_kth
    n_ties = K - jnp.sum(lt.astype(jnp.int32), axis=1, keepdims=True)

    ilo = jnp.zeros((SBLK, 1), jnp.int32)
    ihi = jnp.full((SBLK, 1), n_points - 1, jnp.int32)

    def ibs_body(_, carry):
        ilo, ihi = carry
        mid = ilo + (ihi - ilo) // 2
        cnt = jnp.sum((eq & (iota <= mid)).astype(jnp.int32), axis=1,
                      keepdims=True)
        ge = cnt >= n_ties
        return jnp.where(ge, ilo, mid + 1), jnp.where(ge, mid, ihi)

    n_idx_iters = max(1, (n_points - 1).bit_length())
    ilo, ihi = lax.fori_loop(0, n_idx_iters, ibs_body, (ilo, ihi))
    A = (lt | (eq & (iota <= ilo))).astype(f32)

    out_ref[0] = jnp.dot(A, y_scr[...], preferred_element_type=f32) * f32(1.0 / K)


def kernel(input_points, supernode_idxs, W_in, b_in, W1, b1, W2, b2):
    B, N, _ = input_points.shape
    S = supernode_idxs.shape[1]
    dmat, sinmask, valid = _posembed_consts()
    x = input_points.astype(jnp.float32)
    xt = jnp.transpose(x, (0, 2, 1))
    sidx = supernode_idxs.astype(jnp.int32).reshape(B, S, 1)

    grid = (B, S // SBLK)
    out = pl.pallas_call(
        functools.partial(_body, n_points=N),
        grid=grid,
        in_specs=[
            pl.BlockSpec((1, N, NDIM), lambda b, s: (b, 0, 0)),
            pl.BlockSpec((1, NDIM, N), lambda b, s: (b, 0, 0)),
            pl.BlockSpec((1, SBLK, 1), lambda b, s: (b, s, 0)),
            pl.BlockSpec((NDIM, HIDDEN), lambda b, s: (0, 0)),
            pl.BlockSpec((1, HIDDEN), lambda b, s: (0, 0)),
            pl.BlockSpec((1, HIDDEN), lambda b, s: (0, 0)),
            pl.BlockSpec((NDIM, HIDDEN), lambda b, s: (0, 0)),
            pl.BlockSpec((1, HIDDEN), lambda b, s: (0, 0)),
            pl.BlockSpec((HIDDEN, HIDDEN), lambda b, s: (0, 0)),
            pl.BlockSpec((1, HIDDEN), lambda b, s: (0, 0)),
            pl.BlockSpec((HIDDEN, HIDDEN), lambda b, s: (0, 0)),
            pl.BlockSpec((1, HIDDEN), lambda b, s: (0, 0)),
        ],
        out_specs=pl.BlockSpec((1, SBLK, HIDDEN), lambda b, s: (b, s, 0)),
        out_shape=jax.ShapeDtypeStruct((B, S, HIDDEN), jnp.float32),
        scratch_shapes=[pltpu.VMEM((N, HIDDEN), jnp.float32)],
    )(x, xt, sidx, dmat, sinmask, valid,
      W_in, b_in.reshape(1, HIDDEN), W1, b1.reshape(1, HIDDEN),
      W2, b2.reshape(1, HIDDEN))
    return out


# speedup vs baseline: 12.1064x; 1.0809x over previous
import functools
import numpy as np
import jax
import jax.numpy as jnp
from jax import lax
from jax.experimental import pallas as pl
from jax.experimental.pallas import tpu as pltpu

HIDDEN = 256
NDIM = 3
K = 32
SBLK = 512


def _posembed_consts():
    dim_per = HIDDEN // NDIM
    if dim_per % 2 == 1:
        dim_per -= 1
    half = dim_per // 2
    omega = 1.0 / (10000.0 ** (np.arange(half, dtype=np.float32) / half))
    D = np.zeros((NDIM, HIDDEN), dtype=np.float32)
    phase = np.zeros((1, HIDDEN), dtype=np.float32)
    valid = np.zeros((1, HIDDEN), dtype=np.float32)
    for j in range(NDIM * dim_per):
        d, r = j // dim_per, j % dim_per
        w = omega[r] if r < half else omega[r - half]
        D[d, j] = w
        phase[0, j] = 0.0 if r < half else np.float32(np.pi / 2)
        valid[0, j] = 1.0
    return jnp.asarray(D), jnp.asarray(phase), jnp.asarray(valid)


def _body(x_ref, xt_ref, sidx_ref, dmat_ref, sinm_ref, valid_ref,
          win_ref, bin_ref, w1_ref, b1_ref, w2_ref, b2_ref,
          out_ref, y_scr, *, n_points):
    s_blk = pl.program_id(1)
    f32 = jnp.float32

    @pl.when(s_blk == 0)
    def _compute_mlp():
        xx = x_ref[0]
        proj = jnp.dot(xx, win_ref[...], preferred_element_type=f32) + bin_ref[...]
        t = jnp.dot(xx, dmat_ref[...], preferred_element_type=f32)
        pos = jnp.sin(t + sinm_ref[...]) * valid_ref[...]
        h = proj + pos
        h = jnp.dot(h, w1_ref[...], preferred_element_type=f32) + b1_ref[...]
        h = jax.nn.gelu(h)
        y_scr[...] = jnp.dot(h, w2_ref[...], preferred_element_type=f32) + b2_ref[...]

    sidx = sidx_ref[0]
    iota = lax.broadcasted_iota(jnp.int32, (SBLK, n_points), 1)
    onehot_s = (iota == sidx).astype(f32)
    sup = jnp.dot(onehot_s, x_ref[0], preferred_element_type=f32)

    xt = xt_ref[0]
    acc = jnp.zeros((SBLK, n_points), dtype=f32)
    for d in range(NDIM):
        diff = sup[:, d:d + 1] - xt[d:d + 1, :]
        acc = acc + diff * diff

    bits = lax.bitcast_convert_type(acc, jnp.int32)
    lo = jnp.zeros((SBLK, 1), jnp.int32)
    hi = jnp.max(bits, axis=1, keepdims=True)

    def bs_body(_, carry):
        lo, hi = carry
        mid = lo + (hi - lo) // 2
        cnt = jnp.sum((bits <= mid).astype(jnp.int32), axis=1, keepdims=True)
        ge = cnt >= K
        return jnp.where(ge, lo, mid + 1), jnp.where(ge, mid, hi)

    lo, hi = lax.fori_loop(0, 31, bs_body, (lo, hi))
    v_kth = lo
    lt = bits < v_kth
    eq = bits == v_kth
    n_ties = K - jnp.sum(lt.astype(jnp.int32), axis=1, keepdims=True)

    ilo = jnp.zeros((SBLK, 1), jnp.int32)
    ihi = jnp.full((SBLK, 1), n_points - 1, jnp.int32)

    def ibs_body(_, carry):
        ilo, ihi = carry
        mid = ilo + (ihi - ilo) // 2
        cnt = jnp.sum((eq & (iota <= mid)).astype(jnp.int32), axis=1,
                      keepdims=True)
        ge = cnt >= n_ties
        return jnp.where(ge, ilo, mid + 1), jnp.where(ge, mid, ihi)

    n_idx_iters = max(1, (n_points - 1).bit_length())
    ilo, ihi = lax.fori_loop(0, n_idx_iters, ibs_body, (ilo, ihi))
    A = (lt | (eq & (iota <= ilo))).astype(f32)

    out_ref[0] = jnp.dot(A, y_scr[...], preferred_element_type=f32) * f32(1.0 / K)


def kernel(input_points, supernode_idxs, W_in, b_in, W1, b1, W2, b2):
    B, N, _ = input_points.shape
    S = supernode_idxs.shape[1]
    dmat, sinmask, valid = _posembed_consts()
    x = input_points.astype(jnp.float32)
    xt = jnp.transpose(x, (0, 2, 1))
    sidx = supernode_idxs.astype(jnp.int32).reshape(B, S, 1)

    grid = (B, S // SBLK)
    out = pl.pallas_call(
        functools.partial(_body, n_points=N),
        grid=grid,
        in_specs=[
            pl.BlockSpec((1, N, NDIM), lambda b, s: (b, 0, 0)),
            pl.BlockSpec((1, NDIM, N), lambda b, s: (b, 0, 0)),
            pl.BlockSpec((1, SBLK, 1), lambda b, s: (b, s, 0)),
            pl.BlockSpec((NDIM, HIDDEN), lambda b, s: (0, 0)),
            pl.BlockSpec((1, HIDDEN), lambda b, s: (0, 0)),
            pl.BlockSpec((1, HIDDEN), lambda b, s: (0, 0)),
            pl.BlockSpec((NDIM, HIDDEN), lambda b, s: (0, 0)),
            pl.BlockSpec((1, HIDDEN), lambda b, s: (0, 0)),
            pl.BlockSpec((HIDDEN, HIDDEN), lambda b, s: (0, 0)),
            pl.BlockSpec((1, HIDDEN), lambda b, s: (0, 0)),
            pl.BlockSpec((HIDDEN, HIDDEN), lambda b, s: (0, 0)),
            pl.BlockSpec((1, HIDDEN), lambda b, s: (0, 0)),
        ],
        out_specs=pl.BlockSpec((1, SBLK, HIDDEN), lambda b, s: (b, s, 0)),
        out_shape=jax.ShapeDtypeStruct((B, S, HIDDEN), jnp.float32),
        scratch_shapes=[pltpu.VMEM((N, HIDDEN), jnp.float32)],
    )(x, xt, sidx, dmat, sinmask, valid,
      W_in, b_in.reshape(1, HIDDEN), W1, b1.reshape(1, HIDDEN),
      W2, b2.reshape(1, HIDDEN))
    return out


# speedup vs baseline: 16.3275x; 1.3487x over previous
import functools
import numpy as np
import jax
import jax.numpy as jnp
from jax import lax
from jax.experimental import pallas as pl
from jax.experimental.pallas import tpu as pltpu

HIDDEN = 256
NDIM = 3
K = 32
SBLK = 512


def _posembed_consts():
    dim_per = HIDDEN // NDIM
    if dim_per % 2 == 1:
        dim_per -= 1
    half = dim_per // 2
    omega = 1.0 / (10000.0 ** (np.arange(half, dtype=np.float32) / half))
    D = np.zeros((NDIM, HIDDEN), dtype=np.float32)
    phase = np.zeros((1, HIDDEN), dtype=np.float32)
    valid = np.zeros((1, HIDDEN), dtype=np.float32)
    for j in range(NDIM * dim_per):
        d, r = j // dim_per, j % dim_per
        w = omega[r] if r < half else omega[r - half]
        D[d, j] = w
        phase[0, j] = 0.0 if r < half else np.float32(np.pi / 2)
        valid[0, j] = 1.0
    return jnp.asarray(D), jnp.asarray(phase), jnp.asarray(valid)


def _fast_sin(t):
    f32 = jnp.float32
    k = jnp.round(t * f32(0.3183098861837907))
    r = t - k * f32(3.140625)
    r = r - k * f32(9.676535897932795e-04)
    r = r - k * f32(2.8498605570610653e-10)
    s = r * r
    p = f32(-2.3889859e-08)
    p = p * s + f32(2.7525562e-06)
    p = p * s - f32(1.9840874e-04)
    p = p * s + f32(8.3333310e-03)
    p = p * s - f32(1.6666654e-01)
    sinr = r + r * (s * p)
    odd = (k.astype(jnp.int32) & 1) == 1
    return jnp.where(odd, -sinr, sinr)


def _body(x_ref, xt_ref, sidx_ref, dmat_ref, sinm_ref, valid_ref,
          win_ref, bin_ref, w1_ref, b1_ref, w2_ref, b2_ref,
          out_ref, y_scr, *, n_points):
    s_blk = pl.program_id(1)
    f32 = jnp.float32

    @pl.when(s_blk == 0)
    def _compute_mlp():
        xx = x_ref[0]
        proj = jnp.dot(xx, win_ref[...], preferred_element_type=f32) + bin_ref[...]
        t = jnp.dot(xx, dmat_ref[...], preferred_element_type=f32)
        pos = _fast_sin(t + sinm_ref[...]) * valid_ref[...]
        h = proj + pos
        h = jnp.dot(h, w1_ref[...], preferred_element_type=f32) + b1_ref[...]
        h = jax.nn.gelu(h)
        y_scr[...] = jnp.dot(h, w2_ref[...], preferred_element_type=f32) + b2_ref[...]

    sidx = sidx_ref[0]
    iota = lax.broadcasted_iota(jnp.int32, (SBLK, n_points), 1)
    onehot_s = (iota == sidx).astype(f32)
    sup = jnp.dot(onehot_s, x_ref[0], preferred_element_type=f32)

    xt = xt_ref[0]
    acc = jnp.zeros((SBLK, n_points), dtype=f32)
    for d in range(NDIM):
        diff = sup[:, d:d + 1] - xt[d:d + 1, :]
        acc = acc + diff * diff

    bits = lax.bitcast_convert_type(acc, jnp.int32)
    lo = jnp.zeros((SBLK, 1), jnp.int32)
    hi = jnp.max(bits, axis=1, keepdims=True)

    def bs_body(_, carry):
        lo, hi = carry
        mid = lo + (hi - lo) // 2
        cnt = jnp.sum((bits <= mid).astype(jnp.int32), axis=1, keepdims=True)
        ge = cnt >= K
        return jnp.where(ge, lo, mid + 1), jnp.where(ge, mid, hi)

    lo, hi = lax.fori_loop(0, 31, bs_body, (lo, hi))
    v_kth = lo
    lt = bits < v_kth
    eq = bits == v_kth
    n_ties = K - jnp.sum(lt.astype(jnp.int32), axis=1, keepdims=True)

    chunk = 128
    n_chunks = n_points // chunk
    eqf = eq.astype(f32)
    eqr = eqf.reshape(SBLK * n_chunks, chunk)
    tri_in = (lax.broadcasted_iota(jnp.int32, (chunk, chunk), 0)
              <= lax.broadcasted_iota(jnp.int32, (chunk, chunk), 1)).astype(f32)
    pw = jnp.dot(eqr, tri_in, preferred_element_type=f32)
    csum = pw[:, chunk - 1:chunk].reshape(SBLK, n_chunks)
    tri_ex = (lax.broadcasted_iota(jnp.int32, (n_chunks, n_chunks), 0)
              < lax.broadcasted_iota(jnp.int32, (n_chunks, n_chunks), 1)).astype(f32)
    coff = jnp.dot(csum, tri_ex, preferred_element_type=f32)
    ranks = (pw.reshape(SBLK, n_chunks, chunk)
             + coff[:, :, None]).reshape(SBLK, n_points)
    A = (lt | (eq & (ranks <= n_ties.astype(f32)))).astype(f32)

    out_ref[0] = jnp.dot(A, y_scr[...], preferred_element_type=f32) * f32(1.0 / K)


def kernel(input_points, supernode_idxs, W_in, b_in, W1, b1, W2, b2):
    B, N, _ = input_points.shape
    S = supernode_idxs.shape[1]
    dmat, sinmask, valid = _posembed_consts()
    x = input_points.astype(jnp.float32)
    xt = jnp.transpose(x, (0, 2, 1))
    sidx = supernode_idxs.astype(jnp.int32).reshape(B, S, 1)

    grid = (B, S // SBLK)
    out = pl.pallas_call(
        functools.partial(_body, n_points=N),
        grid=grid,
        in_specs=[
            pl.BlockSpec((1, N, NDIM), lambda b, s: (b, 0, 0)),
            pl.BlockSpec((1, NDIM, N), lambda b, s: (b, 0, 0)),
            pl.BlockSpec((1, SBLK, 1), lambda b, s: (b, s, 0)),
            pl.BlockSpec((NDIM, HIDDEN), lambda b, s: (0, 0)),
            pl.BlockSpec((1, HIDDEN), lambda b, s: (0, 0)),
            pl.BlockSpec((1, HIDDEN), lambda b, s: (0, 0)),
            pl.BlockSpec((NDIM, HIDDEN), lambda b, s: (0, 0)),
            pl.BlockSpec((1, HIDDEN), lambda b, s: (0, 0)),
            pl.BlockSpec((HIDDEN, HIDDEN), lambda b, s: (0, 0)),
            pl.BlockSpec((1, HIDDEN), lambda b, s: (0, 0)),
            pl.BlockSpec((HIDDEN, HIDDEN), lambda b, s: (0, 0)),
            pl.BlockSpec((1, HIDDEN), lambda b, s: (0, 0)),
        ],
        out_specs=pl.BlockSpec((1, SBLK, HIDDEN), lambda b, s: (b, s, 0)),
        out_shape=jax.ShapeDtypeStruct((B, S, HIDDEN), jnp.float32),
        scratch_shapes=[pltpu.VMEM((N, HIDDEN), jnp.float32)],
    )(x, xt, sidx, dmat, sinmask, valid,
      W_in, b_in.reshape(1, HIDDEN), W1, b1.reshape(1, HIDDEN),
      W2, b2.reshape(1, HIDDEN))
    return out


# speedup vs baseline: 16.5291x; 1.0123x over previous
import functools
import numpy as np
import jax
import jax.numpy as jnp
from jax import lax
from jax.experimental import pallas as pl
from jax.experimental.pallas import tpu as pltpu

HIDDEN = 256
NDIM = 3
K = 32
SBLK = 1024


def _posembed_consts():
    dim_per = HIDDEN // NDIM
    if dim_per % 2 == 1:
        dim_per -= 1
    half = dim_per // 2
    omega = 1.0 / (10000.0 ** (np.arange(half, dtype=np.float32) / half))
    D = np.zeros((NDIM, HIDDEN), dtype=np.float32)
    phase = np.zeros((1, HIDDEN), dtype=np.float32)
    valid = np.zeros((1, HIDDEN), dtype=np.float32)
    for j in range(NDIM * dim_per):
        d, r = j // dim_per, j % dim_per
        w = omega[r] if r < half else omega[r - half]
        D[d, j] = w
        phase[0, j] = 0.0 if r < half else np.float32(np.pi / 2)
        valid[0, j] = 1.0
    return jnp.asarray(D), jnp.asarray(phase), jnp.asarray(valid)


def _fast_sin(t):
    f32 = jnp.float32
    k = jnp.round(t * f32(0.3183098861837907))
    r = t - k * f32(3.140625)
    r = r - k * f32(9.676535897932795e-04)
    r = r - k * f32(2.8498605570610653e-10)
    s = r * r
    p = f32(-2.3889859e-08)
    p = p * s + f32(2.7525562e-06)
    p = p * s - f32(1.9840874e-04)
    p = p * s + f32(8.3333310e-03)
    p = p * s - f32(1.6666654e-01)
    sinr = r + r * (s * p)
    odd = (k.astype(jnp.int32) & 1) == 1
    return jnp.where(odd, -sinr, sinr)


def _body(x_ref, xt_ref, sidx_ref, dmat_ref, sinm_ref, valid_ref,
          win_ref, bin_ref, w1_ref, b1_ref, w2_ref, b2_ref,
          out_ref, y_scr, *, n_points):
    s_blk = pl.program_id(1)
    f32 = jnp.float32

    @pl.when(s_blk == 0)
    def _compute_mlp():
        xx = x_ref[0]
        proj = jnp.dot(xx, win_ref[...], preferred_element_type=f32) + bin_ref[...]
        t = jnp.dot(xx, dmat_ref[...], preferred_element_type=f32)
        pos = _fast_sin(t + sinm_ref[...]) * valid_ref[...]
        h = proj + pos
        h = jnp.dot(h, w1_ref[...], preferred_element_type=f32) + b1_ref[...]
        h = jax.nn.gelu(h)
        y_scr[...] = jnp.dot(h, w2_ref[...], preferred_element_type=f32) + b2_ref[...]

    sidx = sidx_ref[0]
    iota = lax.broadcasted_iota(jnp.int32, (SBLK, n_points), 1)
    onehot_s = (iota == sidx).astype(f32)
    sup = jnp.dot(onehot_s, x_ref[0], preferred_element_type=f32)

    xt = xt_ref[0]
    acc = jnp.zeros((SBLK, n_points), dtype=f32)
    for d in range(NDIM):
        diff = sup[:, d:d + 1] - xt[d:d + 1, :]
        acc = acc + diff * diff

    bits = lax.bitcast_convert_type(acc, jnp.int32)
    lo = jnp.zeros((SBLK, 1), jnp.int32)
    hi = jnp.max(bits, axis=1, keepdims=True)

    def bs_body(_, carry):
        lo, hi = carry
        mid = lo + (hi - lo) // 2
        cnt = jnp.sum((bits <= mid).astype(jnp.int32), axis=1, keepdims=True)
        ge = cnt >= K
        return jnp.where(ge, lo, mid + 1), jnp.where(ge, mid, hi)

    lo, hi = lax.fori_loop(0, 31, bs_body, (lo, hi))
    v_kth = lo
    lt = bits < v_kth
    eq = bits == v_kth
    n_ties = K - jnp.sum(lt.astype(jnp.int32), axis=1, keepdims=True)

    chunk = 128
    n_chunks = n_points // chunk
    eqf = eq.astype(f32)
    eqr = eqf.reshape(SBLK * n_chunks, chunk)
    tri_in = (lax.broadcasted_iota(jnp.int32, (chunk, chunk), 0)
              <= lax.broadcasted_iota(jnp.int32, (chunk, chunk), 1)).astype(f32)
    pw = jnp.dot(eqr, tri_in, preferred_element_type=f32)
    csum = pw[:, chunk - 1:chunk].reshape(SBLK, n_chunks)
    tri_ex = (lax.broadcasted_iota(jnp.int32, (n_chunks, n_chunks), 0)
              < lax.broadcasted_iota(jnp.int32, (n_chunks, n_chunks), 1)).astype(f32)
    coff = jnp.dot(csum, tri_ex, preferred_element_type=f32)
    ranks = (pw.reshape(SBLK, n_chunks, chunk)
             + coff[:, :, None]).reshape(SBLK, n_points)
    A = (lt | (eq & (ranks <= n_ties.astype(f32)))).astype(f32)

    out_ref[0] = jnp.dot(A, y_scr[...], preferred_element_type=f32) * f32(1.0 / K)


def kernel(input_points, supernode_idxs, W_in, b_in, W1, b1, W2, b2):
    B, N, _ = input_points.shape
    S = supernode_idxs.shape[1]
    dmat, sinmask, valid = _posembed_consts()
    x = input_points.astype(jnp.float32)
    xt = jnp.transpose(x, (0, 2, 1))
    sidx = supernode_idxs.astype(jnp.int32).reshape(B, S, 1)

    grid = (B, S // SBLK)
    out = pl.pallas_call(
        functools.partial(_body, n_points=N),
        grid=grid,
        in_specs=[
            pl.BlockSpec((1, N, NDIM), lambda b, s: (b, 0, 0)),
            pl.BlockSpec((1, NDIM, N), lambda b, s: (b, 0, 0)),
            pl.BlockSpec((1, SBLK, 1), lambda b, s: (b, s, 0)),
            pl.BlockSpec((NDIM, HIDDEN), lambda b, s: (0, 0)),
            pl.BlockSpec((1, HIDDEN), lambda b, s: (0, 0)),
            pl.BlockSpec((1, HIDDEN), lambda b, s: (0, 0)),
            pl.BlockSpec((NDIM, HIDDEN), lambda b, s: (0, 0)),
            pl.BlockSpec((1, HIDDEN), lambda b, s: (0, 0)),
            pl.BlockSpec((HIDDEN, HIDDEN), lambda b, s: (0, 0)),
            pl.BlockSpec((1, HIDDEN), lambda b, s: (0, 0)),
            pl.BlockSpec((HIDDEN, HIDDEN), lambda b, s: (0, 0)),
            pl.BlockSpec((1, HIDDEN), lambda b, s: (0, 0)),
        ],
        out_specs=pl.BlockSpec((1, SBLK, HIDDEN), lambda b, s: (b, s, 0)),
        out_shape=jax.ShapeDtypeStruct((B, S, HIDDEN), jnp.float32),
        scratch_shapes=[pltpu.VMEM((N, HIDDEN), jnp.float32)],
    )(x, xt, sidx, dmat, sinmask, valid,
      W_in, b_in.reshape(1, HIDDEN), W1, b1.reshape(1, HIDDEN),
      W2, b2.reshape(1, HIDDEN))
    return out
